# baseline (speedup 1.0000x reference)
import functools

import jax
import jax.numpy as jnp
from jax import lax
from jax.experimental import pallas as pl
from jax.experimental.pallas import tpu as pltpu
from jax.experimental.pallas import tpu_sc as plsc

B = 16384
N_CONT = 13
N_CAT = 26
VOCAB = 100000
D = 32
N_TGT = 2

NC, NS = 2, 16
NW = NC * NS
ROWS = N_CAT * D
ROWS_PER_W = ROWS // NW
CONT_ROWS = N_CONT * D
CTX_ROWS = CONT_ROWS + ROWS
SPLIT = 50048
SEG_B = VOCAB - SPLIT
RQ = B // 4
NRB = 3


def _sc_gather_t(tabT, xcT_flat):
    mesh = plsc.VectorSubcoreMesh(core_axis_name="c", subcore_axis_name="s")

    @functools.partial(
        pl.kernel,
        mesh=mesh,
        compiler_params=pltpu.CompilerParams(
            use_tc_tiling_on_sc=True, needs_layout_passes=False),
        out_type=jax.ShapeDtypeStruct((CTX_ROWS, B), jnp.float32),
        scratch_types=[
            pltpu.VMEM((1, SPLIT), jnp.float32),
            pltpu.VMEM((1, SEG_B), jnp.float32),
            pltpu.VMEM((B,), jnp.int32),
            pltpu.VMEM((NRB, 1, RQ), jnp.float32),
            pltpu.SemaphoreType.DMA,
            pltpu.SemaphoreType.DMA,
            pltpu.SemaphoreType.DMA((NRB,)),
        ],
    )
    def k(tab_hbm, xc_hbm, out_hbm, seg_a, seg_b, idx_v, row_b,
          sem_a, sem_b, sem_w):
        wid = lax.axis_index("s") * NC + lax.axis_index("c")
        zeros16 = jnp.zeros((16,), jnp.int32)
        iota16 = jax.lax.iota(jnp.int32, 16)

        for r in range(NRB):
            pltpu.async_copy(
                tab_hbm.at[pl.ds(0, 1), pl.ds(0, RQ)],
                row_b.at[r], sem_w.at[r])

        def pair_body(p, prev_f):
            fd = wid * ROWS_PER_W + p
            f = fd // D

            @pl.when(f != prev_f)
            def _load_idx():
                pltpu.sync_copy(xc_hbm.at[pl.ds(f * B, B)], idx_v)

            cp_a = pltpu.async_copy(
                tab_hbm.at[pl.ds(fd, 1), pl.ds(0, SPLIT)], seg_a, sem_a)
            cp_b = pltpu.async_copy(
                tab_hbm.at[pl.ds(fd, 1), pl.ds(SPLIT, SEG_B)], seg_b, sem_b)
            cp_a.wait()

            for q in range(4):
                r = q % NRB
                pltpu.make_async_copy(
                    tab_hbm.at[pl.ds(0, 1), pl.ds(0, RQ)],
                    row_b.at[r], sem_w.at[r]).wait()

                def pass_a(j, c2):
                    idx16 = idx_v[pl.ds((q * (RQ // 16) + j) * 16, 16)]
                    m = idx16 < SPLIT
                    vals = plsc.load_gather(seg_a, [zeros16, idx16], mask=m)
                    plsc.store_scatter(
                        row_b.at[r], [zeros16, iota16 + j * 16], vals, mask=m)
                    return c2

                lax.fori_loop(0, RQ // 16, pass_a, 0, unroll=8)
                if q == 0:
                    cp_b.wait()

                def pass_b(j, c2):
                    idx16 = idx_v[pl.ds((q * (RQ // 16) + j) * 16, 16)]
                    m = idx16 >= SPLIT
                    vals = plsc.load_gather(
                        seg_b, [zeros16, idx16 - SPLIT], mask=m)
                    plsc.store_scatter(
                        row_b.at[r], [zeros16, iota16 + j * 16], vals, mask=m)
                    return c2

                lax.fori_loop(0, RQ // 16, pass_b, 0, unroll=8)
                pltpu.async_copy(
                    row_b.at[r],
                    out_hbm.at[pl.ds(CONT_ROWS + fd, 1),
                               pl.ds(q * RQ, RQ)],
                    sem_w.at[r])
            return f

        lax.fori_loop(0, ROWS_PER_W, pair_body, -1, unroll=False)

        for r in range(NRB):
            pltpu.make_async_copy(
                tab_hbm.at[pl.ds(0, 1), pl.ds(0, RQ)],
                row_b.at[r], sem_w.at[r]).wait()

    return k(tabT, xcT_flat)


BC = 2048


def _tc_body(x_ref, wt_ref, bt_ref, tgtT_ref, ctx_in_ref, ctx_ref, cls_ref):
    del ctx_in_ref
    for i in range(N_CONT):
        ctx_ref[pl.ds(i * D, D), :] = (
            wt_ref[:, i:i + 1] * x_ref[i:i + 1, :] + bt_ref[:, i:i + 1])
    for t in range(N_TGT):
        cls_ref[pl.ds(t * D, D), :] = jnp.broadcast_to(
            tgtT_ref[:, t:t + 1], (D, BC))


def _tc_assemble(x_contT, WT, bT, tgtT, ctx_partial):
    grid = (B // BC,)
    ctxT, clsT = pl.pallas_call(
        _tc_body,
        grid=grid,
        in_specs=[
            pl.BlockSpec((N_CONT, BC), lambda i: (0, i)),
            pl.BlockSpec((D, N_CONT), lambda i: (0, 0)),
            pl.BlockSpec((D, N_CONT), lambda i: (0, 0)),
            pl.BlockSpec((D, N_TGT), lambda i: (0, 0)),
            pl.BlockSpec((8, 128), lambda i: (0, 0)),
        ],
        out_specs=[
            pl.BlockSpec((CONT_ROWS, BC), lambda i: (0, i)),
            pl.BlockSpec((N_TGT * D, BC), lambda i: (0, i)),
        ],
        out_shape=[
            jax.ShapeDtypeStruct((CTX_ROWS, B), jnp.float32),
            jax.ShapeDtypeStruct((N_TGT * D, B), jnp.float32),
        ],
        input_output_aliases={4: 0},
    )(x_contT, WT, bT, tgtT, ctx_partial)
    return ctxT, clsT


def kernel(x_cat, x_cont, lin_W, lin_b, cat_tables, tgt):
    tabT = jnp.transpose(cat_tables, (0, 2, 1)).reshape(ROWS, VOCAB)
    xcT_flat = jnp.transpose(x_cat, (1, 0)).reshape(N_CAT * B)
    x_contT = jnp.transpose(x_cont, (1, 0))
    WT = jnp.transpose(lin_W, (1, 0))
    bT = jnp.transpose(lin_b, (1, 0))
    tgtT = jnp.transpose(tgt, (1, 0))

    ctx_partial = _sc_gather_t(tabT, xcT_flat)
    ctxT, clsT = _tc_assemble(x_contT, WT, bT, tgtT, ctx_partial)

    context = jnp.transpose(
        ctxT.reshape(N_CONT + N_CAT, D, B), (2, 0, 1))
    class_embeddings = jnp.transpose(clsT.reshape(N_TGT, D, B), (2, 0, 1))
    return (class_embeddings, context)


# speedup vs baseline: 1.8087x; 1.8087x over previous
import functools

import jax
import jax.numpy as jnp
from jax import lax
from jax.experimental import pallas as pl
from jax.experimental.pallas import tpu as pltpu
from jax.experimental.pallas import tpu_sc as plsc

B = 16384
N_CONT = 13
N_CAT = 26
VOCAB = 100000
D = 32
N_TGT = 2

NC, NS = 2, 16
NW = NC * NS
ROWS = N_CAT * D
ROWS_PER_W = ROWS // NW
CONT_ROWS = N_CONT * D
CTX_ROWS = CONT_ROWS + ROWS
RQ = B // 4
NRB = 3


def _sc_gather_t(tabT, xcT_flat):
    mesh = plsc.VectorSubcoreMesh(core_axis_name="c", subcore_axis_name="s")

    @functools.partial(
        pl.kernel,
        mesh=mesh,
        compiler_params=pltpu.CompilerParams(
            use_tc_tiling_on_sc=True, needs_layout_passes=False),
        out_type=jax.ShapeDtypeStruct((CTX_ROWS, B), jnp.float32),
        scratch_types=[
            pltpu.VMEM((1, VOCAB), jnp.float32),
            pltpu.VMEM((B,), jnp.int32),
            pltpu.VMEM((NRB, 1, RQ), jnp.float32),
            pltpu.SemaphoreType.DMA((NRB,)),
        ],
    )
    def k(tab_hbm, xc_hbm, out_hbm, seg_v, idx_v, row_b, sem_w):
        wid = lax.axis_index("s") * NC + lax.axis_index("c")
        zeros16 = jnp.zeros((16,), jnp.int32)

        for r in range(NRB):
            pltpu.async_copy(
                tab_hbm.at[pl.ds(0, 1), pl.ds(0, RQ)],
                row_b.at[r], sem_w.at[r])

        def pair_body(p, prev_f):
            fd = wid * ROWS_PER_W + p
            f = fd // D

            @pl.when(f != prev_f)
            def _load_idx():
                pltpu.sync_copy(xc_hbm.at[pl.ds(f * B, B)], idx_v)

            pltpu.sync_copy(tab_hbm.at[pl.ds(fd, 1), :], seg_v)

            for q in range(4):
                r = q % NRB
                pltpu.make_async_copy(
                    tab_hbm.at[pl.ds(0, 1), pl.ds(0, RQ)],
                    row_b.at[r], sem_w.at[r]).wait()

                def vec_body(j, c2):
                    idx16 = idx_v[pl.ds((q * (RQ // 16) + j) * 16, 16)]
                    vals = plsc.load_gather(seg_v, [zeros16, idx16])
                    row_b.at[r][0, pl.ds(j * 16, 16)] = vals
                    return c2

                lax.fori_loop(0, RQ // 16, vec_body, 0, unroll=8)
                pltpu.async_copy(
                    row_b.at[r],
                    out_hbm.at[pl.ds(CONT_ROWS + fd, 1),
                               pl.ds(q * RQ, RQ)],
                    sem_w.at[r])
            return f

        lax.fori_loop(0, ROWS_PER_W, pair_body, -1, unroll=False)

        for r in range(NRB):
            pltpu.make_async_copy(
                tab_hbm.at[pl.ds(0, 1), pl.ds(0, RQ)],
                row_b.at[r], sem_w.at[r]).wait()

    return k(tabT, xcT_flat)


BC = 2048


def _tc_body(x_ref, wt_ref, bt_ref, tgtT_ref, ctx_in_ref, ctx_ref, cls_ref):
    del ctx_in_ref
    for i in range(N_CONT):
        ctx_ref[pl.ds(i * D, D), :] = (
            wt_ref[:, i:i + 1] * x_ref[i:i + 1, :] + bt_ref[:, i:i + 1])
    for t in range(N_TGT):
        cls_ref[pl.ds(t * D, D), :] = jnp.broadcast_to(
            tgtT_ref[:, t:t + 1], (D, BC))


def _tc_assemble(x_contT, WT, bT, tgtT, ctx_partial):
    grid = (B // BC,)
    ctxT, clsT = pl.pallas_call(
        _tc_body,
        grid=grid,
        in_specs=[
            pl.BlockSpec((N_CONT, BC), lambda i: (0, i)),
            pl.BlockSpec((D, N_CONT), lambda i: (0, 0)),
            pl.BlockSpec((D, N_CONT), lambda i: (0, 0)),
            pl.BlockSpec((D, N_TGT), lambda i: (0, 0)),
            pl.BlockSpec((8, 128), lambda i: (0, 0)),
        ],
        out_specs=[
            pl.BlockSpec((CONT_ROWS, BC), lambda i: (0, i)),
            pl.BlockSpec((N_TGT * D, BC), lambda i: (0, i)),
        ],
        out_shape=[
            jax.ShapeDtypeStruct((CTX_ROWS, B), jnp.float32),
            jax.ShapeDtypeStruct((N_TGT * D, B), jnp.float32),
        ],
        input_output_aliases={4: 0},
    )(x_contT, WT, bT, tgtT, ctx_partial)
    return ctxT, clsT


def kernel(x_cat, x_cont, lin_W, lin_b, cat_tables, tgt):
    tabT = jnp.transpose(cat_tables, (0, 2, 1)).reshape(ROWS, VOCAB)
    xcT_flat = jnp.transpose(x_cat, (1, 0)).reshape(N_CAT * B)
    x_contT = jnp.transpose(x_cont, (1, 0))
    WT = jnp.transpose(lin_W, (1, 0))
    bT = jnp.transpose(lin_b, (1, 0))
    tgtT = jnp.transpose(tgt, (1, 0))

    ctx_partial = _sc_gather_t(tabT, xcT_flat)
    ctxT, clsT = _tc_assemble(x_contT, WT, bT, tgtT, ctx_partial)

    context = jnp.transpose(
        ctxT.reshape(N_CONT + N_CAT, D, B), (2, 0, 1))
    class_embeddings = jnp.transpose(clsT.reshape(N_TGT, D, B), (2, 0, 1))
    return (class_embeddings, context)
